# Initial kernel scaffold; baseline (speedup 1.0000x reference)
import jax
import jax.numpy as jnp
from jax.experimental import pallas as pl


def kernel(cur_state, input_image):
    raise NotImplementedError("write your pallas kernel here")


# speedup vs baseline: 1.0434x; 1.0434x over previous
import jax
import jax.numpy as jnp
from jax.experimental import pallas as pl

_BILATERAL = True
_THETA_ALPHA = 8.0
_THETA_BETA = 0.125
_THETA_GAMMA = 3.0

_BI = 256


def _gauss_tile(f_blk_ref, f_all_ref, v_ref, out_ref):
    fi = f_blk_ref[0]
    fa = f_all_ref[0]
    vb = v_ref[0]
    sqi = jnp.sum(fi * fi, axis=0)
    sqa = jnp.sum(fa * fa, axis=0)
    g = jax.lax.dot_general(
        fi, fa, (((0,), (0,)), ((), ())),
        preferred_element_type=jnp.float32)
    d2 = (sqi[:, None] + sqa[None, :]) - 2.0 * g
    k_mat = jnp.exp(-0.5 * jnp.maximum(d2, 0.0))
    out_ref[0] = jax.lax.dot_general(
        vb, k_mat, (((1,), (1,)), ((), ())),
        preferred_element_type=jnp.float32)


@jax.jit
def kernel(cur_state, input_image):
    B, C, H, W = cur_state.shape
    N = H * W

    yy = jax.lax.broadcasted_iota(jnp.float32, (H, W), 0)
    xx = jax.lax.broadcasted_iota(jnp.float32, (H, W), 1)
    if _BILATERAL:
        pos = jnp.stack([xx, yy], axis=0).reshape(2, N) / _THETA_ALPHA
        col = input_image.reshape(B, 3, N) / _THETA_BETA
        feats = jnp.concatenate(
            [jnp.broadcast_to(pos[None], (B, 2, N)), col,
             jnp.zeros((B, 3, N), jnp.float32)], axis=1)
    else:
        pos = jnp.stack([xx, yy], axis=0).reshape(2, N) / _THETA_GAMMA
        feats = jnp.concatenate(
            [jnp.broadcast_to(pos[None], (B, 2, N)),
             jnp.zeros((B, 6, N), jnp.float32)], axis=1)

    v = cur_state.reshape(B, C, N)

    out = pl.pallas_call(
        _gauss_tile,
        grid=(B, N // _BI),
        in_specs=[
            pl.BlockSpec((1, 8, _BI), lambda b, i: (b, 0, i)),
            pl.BlockSpec((1, 8, N), lambda b, i: (b, 0, 0)),
            pl.BlockSpec((1, C, N), lambda b, i: (b, 0, 0)),
        ],
        out_specs=pl.BlockSpec((1, C, _BI), lambda b, i: (b, 0, i)),
        out_shape=jax.ShapeDtypeStruct((B, C, N), jnp.float32),
    )(feats, feats, v)

    return out.reshape(B, C, H, W)


# speedup vs baseline: 1.0536x; 1.0097x over previous
import jax
import jax.numpy as jnp
from jax.experimental import pallas as pl

_BILATERAL = True
_THETA_ALPHA = 8.0
_THETA_BETA = 0.125
_THETA_GAMMA = 3.0

_BI = 256
_LOG2E = 1.4426950408889634


_NHALF_LOG2E = -0.5 * _LOG2E


def _gauss_tile(f_blk_ref, f_all_ref, v_ref, out_ref):
    fi = f_blk_ref[0]
    fa = f_all_ref[0]
    vb = v_ref[0]
    sqi = jnp.sum(fi * fi, axis=0)
    sqa = jnp.sum(fa * fa, axis=0)
    g = jax.lax.dot_general(
        fi, fa, (((0,), (0,)), ((), ())),
        preferred_element_type=jnp.float32)
    d2 = (sqi[:, None] + sqa[None, :]) - 2.0 * g
    k_mat = jnp.exp2(_NHALF_LOG2E * jnp.maximum(d2, 0.0))
    out_ref[0] = jax.lax.dot_general(
        vb, k_mat, (((1,), (1,)), ((), ())),
        preferred_element_type=jnp.float32)


@jax.jit
def kernel(cur_state, input_image):
    B, C, H, W = cur_state.shape
    N = H * W

    yy = jax.lax.broadcasted_iota(jnp.float32, (H, W), 0)
    xx = jax.lax.broadcasted_iota(jnp.float32, (H, W), 1)
    if _BILATERAL:
        pos = jnp.stack([xx, yy], axis=0).reshape(2, N) / _THETA_ALPHA
        col = input_image.reshape(B, 3, N) / _THETA_BETA
        feats = jnp.concatenate(
            [jnp.broadcast_to(pos[None], (B, 2, N)), col,
             jnp.zeros((B, 3, N), jnp.float32)], axis=1)
    else:
        pos = jnp.stack([xx, yy], axis=0).reshape(2, N) / _THETA_GAMMA
        feats = jnp.concatenate(
            [jnp.broadcast_to(pos[None], (B, 2, N)),
             jnp.zeros((B, 6, N), jnp.float32)], axis=1)

    v = cur_state.reshape(B, C, N)

    out = pl.pallas_call(
        _gauss_tile,
        grid=(B, N // _BI),
        in_specs=[
            pl.BlockSpec((1, 8, _BI), lambda b, i: (b, 0, i)),
            pl.BlockSpec((1, 8, N), lambda b, i: (b, 0, 0)),
            pl.BlockSpec((1, C, N), lambda b, i: (b, 0, 0)),
        ],
        out_specs=pl.BlockSpec((1, C, _BI), lambda b, i: (b, 0, i)),
        out_shape=jax.ShapeDtypeStruct((B, C, N), jnp.float32),
    )(feats, feats, v)

    return out.reshape(B, C, H, W)
